# Initial kernel scaffold; baseline (speedup 1.0000x reference)
import jax
import jax.numpy as jnp
from jax.experimental import pallas as pl


def kernel(x, edge_index, batch, scatter_edge_index, scatter_edge_attr, W_in, b_in, W1, b1, W2, b2):
    raise NotImplementedError("write your pallas kernel here")


# speedup vs baseline: 4.7969x; 4.7969x over previous
import functools

import jax
import jax.numpy as jnp
from jax import lax
from jax.experimental import pallas as pl
from jax.experimental.pallas import tpu as pltpu
from jax.experimental.pallas import tpu_sc as plsc

_N = 10000
_E = 320000
_D = 128

_NC = 2
_NS = 16
_NW = _NC * _NS
_CH = 80
_EPT = _E // _NW
_NCHUNK = _EPT // _CH
_RCH = 80
_NRCH = _N // _RCH


def _sc_spmm_body(h_hbm, edges_hbm, attr_hbm, out_hbm, ed_v, attr_v, msg_v,
                  acc_sh, sem):
    cid = lax.axis_index("c")
    sid = lax.axis_index("s")
    wid = cid * _NS + sid

    def _zrow(i, c):
        for k in range(_D // 16):
            msg_v[i, pl.ds(k * 16, 16)] = jnp.zeros((16,), jnp.float32)
        return c
    lax.fori_loop(0, _RCH, _zrow, 0)

    for t in range(-(-_NRCH // _NS)):
        rc = sid + t * _NS
        @pl.when(rc < _NRCH)
        def _():
            pltpu.sync_copy(msg_v, acc_sh.at[pl.ds(rc * _RCH, _RCH)])
    plsc.subcore_barrier()

    pltpu.sync_copy(attr_hbm.at[wid], attr_v)

    def _chunk(ci, c):
        pltpu.sync_copy(edges_hbm.at[wid, ci], ed_v)
        pltpu.async_copy(h_hbm.at[ed_v.at[0]], msg_v, sem).wait()

        def _grp(gi, cc):
            av = attr_v[ci, pl.ds(gi * 16, 16)]
            for j in range(16):
                a = av[j]
                e = gi * 16 + j
                for k in range(_D // 16):
                    sl = pl.ds(k * 16, 16)
                    msg_v[e, sl] = msg_v[e, sl] * a
            return cc
        lax.fori_loop(0, _CH // 16, _grp, 0)
        pltpu.sync_copy(msg_v, acc_sh.at[ed_v.at[1]], add=True)
        return c
    lax.fori_loop(0, _NCHUNK, _chunk, 0)
    plsc.subcore_barrier()

    for t in range(-(-_NRCH // _NS)):
        rc = sid + t * _NS
        @pl.when(rc < _NRCH)
        def _():
            pltpu.sync_copy(acc_sh.at[pl.ds(rc * _RCH, _RCH)],
                            out_hbm.at[cid, pl.ds(rc * _RCH, _RCH)])


_sc_spmm = pl.kernel(
    _sc_spmm_body,
    out_type=jax.ShapeDtypeStruct((_NC, _N, _D), jnp.float32),
    mesh=plsc.VectorSubcoreMesh(core_axis_name="c", subcore_axis_name="s"),
    scratch_types=[
        pltpu.VMEM((2, _CH), jnp.int32),
        pltpu.VMEM((_NCHUNK, _CH), jnp.float32),
        pltpu.VMEM((_CH, _D), jnp.float32),
        pltpu.VMEM_SHARED((_N, _D), jnp.float32),
        pltpu.SemaphoreType.DMA,
    ],
)


_RB = 2000
_GRID = _N // _RB


def _linear_body(x_ref, w_ref, b_ref, o_ref):
    o_ref[...] = (jnp.dot(x_ref[...], w_ref[...],
                          preferred_element_type=jnp.float32) + b_ref[...])


def _tc_linear(x, w, b):
    return pl.pallas_call(
        _linear_body,
        grid=(_GRID,),
        in_specs=[pl.BlockSpec((_RB, _D), lambda i: (i, 0)),
                  pl.BlockSpec((_D, _D), lambda i: (0, 0)),
                  pl.BlockSpec((1, _D), lambda i: (0, 0))],
        out_specs=pl.BlockSpec((_RB, _D), lambda i: (i, 0)),
        out_shape=jax.ShapeDtypeStruct((_N, _D), jnp.float32),
    )(x, w, b.reshape(1, _D))


def _mlp(g, w1, b1, w2, b2):
    t = jnp.maximum(jnp.dot(g, w1, preferred_element_type=jnp.float32) + b1,
                    0.0)
    return jnp.dot(t, w2, preferred_element_type=jnp.float32) + b2


def _res_body(g_ref, h_ref, w1_ref, b1_ref, w2_ref, b2_ref, o_ref):
    g = g_ref[0] + g_ref[1]
    o_ref[...] = h_ref[...] + _mlp(g, w1_ref[...], b1_ref[...],
                                   w2_ref[...], b2_ref[...])


def _plain_body(g_ref, w1_ref, b1_ref, w2_ref, b2_ref, o_ref):
    g = g_ref[0] + g_ref[1]
    o_ref[...] = _mlp(g, w1_ref[...], b1_ref[...], w2_ref[...], b2_ref[...])


def _dual_body(g_ref, h_ref, w1a_ref, b1a_ref, w2a_ref, b2a_ref,
               w1b_ref, b1b_ref, w2b_ref, b2b_ref, s_ref, hn_ref):
    g = g_ref[0] + g_ref[1]
    s_ref[...] = _mlp(g, w1a_ref[...], b1a_ref[...], w2a_ref[...], b2a_ref[...])
    hn_ref[...] = h_ref[...] + _mlp(g, w1b_ref[...], b1b_ref[...],
                                    w2b_ref[...], b2b_ref[...])


_g_spec = pl.BlockSpec((_NC, _RB, _D), lambda i: (0, i, 0))
_row_spec = pl.BlockSpec((_RB, _D), lambda i: (i, 0))
_w_spec = pl.BlockSpec((_D, _D), lambda i: (0, 0))
_b_spec = pl.BlockSpec((1, _D), lambda i: (0, 0))
_row_shape = jax.ShapeDtypeStruct((_N, _D), jnp.float32)


def _tc_res(g, h, w1, b1, w2, b2):
    return pl.pallas_call(
        _res_body,
        grid=(_GRID,),
        in_specs=[_g_spec, _row_spec, _w_spec, _b_spec, _w_spec, _b_spec],
        out_specs=_row_spec,
        out_shape=_row_shape,
    )(g, h, w1, b1.reshape(1, _D), w2, b2.reshape(1, _D))


def _tc_plain(g, w1, b1, w2, b2):
    return pl.pallas_call(
        _plain_body,
        grid=(_GRID,),
        in_specs=[_g_spec, _w_spec, _b_spec, _w_spec, _b_spec],
        out_specs=_row_spec,
        out_shape=_row_shape,
    )(g, w1, b1.reshape(1, _D), w2, b2.reshape(1, _D))


def _tc_dual(g, h, w1a, b1a, w2a, b2a, w1b, b1b, w2b, b2b):
    return pl.pallas_call(
        _dual_body,
        grid=(_GRID,),
        in_specs=[_g_spec, _row_spec,
                  _w_spec, _b_spec, _w_spec, _b_spec,
                  _w_spec, _b_spec, _w_spec, _b_spec],
        out_specs=(_row_spec, _row_spec),
        out_shape=(_row_shape, _row_shape),
    )(g, h, w1a, b1a.reshape(1, _D), w2a, b2a.reshape(1, _D),
      w1b, b1b.reshape(1, _D), w2b, b2b.reshape(1, _D))


def kernel(x, edge_index, batch, scatter_edge_index, scatter_edge_attr,
           W_in, b_in, W1, b1, W2, b2):
    edges = jnp.stack(
        [scatter_edge_index[0].reshape(_NW, _NCHUNK, _CH),
         scatter_edge_index[1].reshape(_NW, _NCHUNK, _CH)], axis=2)
    attr = scatter_edge_attr.reshape(_NW, _NCHUNK, _CH)

    h1 = _tc_linear(x, W_in, b_in)
    g1 = _sc_spmm(h1, edges, attr)
    h2 = _tc_res(g1, h1, W1[0], b1[0], W2[0], b2[0])
    g2 = _sc_spmm(h2, edges, attr)
    s0, h3 = _tc_dual(g2, h2, W1[0], b1[0], W2[0], b2[0],
                      W1[1], b1[1], W2[1], b2[1])
    g3 = _sc_spmm(h3, edges, attr)
    s1, h4 = _tc_dual(g3, h3, W1[1], b1[1], W2[1], b2[1],
                      W1[2], b1[2], W2[2], b2[2])
    g4 = _sc_spmm(h4, edges, attr)
    s2 = _tc_plain(g4, W1[2], b1[2], W2[2], b2[2])
    return ((s0, s1, s2), (h2, h3, h4))


# speedup vs baseline: 7.0582x; 1.4714x over previous
import functools

import jax
import jax.numpy as jnp
from jax import lax
from jax.experimental import pallas as pl
from jax.experimental.pallas import tpu as pltpu
from jax.experimental.pallas import tpu_sc as plsc

_N = 10000
_E = 320000
_D = 128

_NC = 2
_NS = 16
_NW = _NC * _NS
_CH = 80
_EPT = _E // _NW
_NCHUNK = _EPT // _CH
_RCH = 80
_NRCH = _N // _RCH


def _sc_spmm_body(h_hbm, edges_hbm, attr_hbm, out_hbm,
                  ed0, ed1, at0, at1, dv0, dv1, msg0, msg1, acc_sh,
                  sg0, sg1, ss0, ss1, sp0, sp1):
    cid = lax.axis_index("c")
    sid = lax.axis_index("s")
    wid = cid * _NS + sid

    eds = (ed0, ed1)
    ats = (at0, at1)
    dvs = (dv0, dv1)
    msgs = (msg0, msg1)
    sgs = (sg0, sg1)
    sss = (ss0, ss1)
    sps = (sp0, sp1)

    def _zrow(i, c):
        for k in range(_D // 16):
            msg0[i, pl.ds(k * 16, 16)] = jnp.zeros((16,), jnp.float32)
        return c
    lax.fori_loop(0, _RCH, _zrow, 0)

    for t in range(-(-_NRCH // _NS)):
        rc = sid + t * _NS
        @pl.when(rc < _NRCH)
        def _():
            pltpu.sync_copy(msg0, acc_sh.at[pl.ds(rc * _RCH, _RCH)])
    plsc.subcore_barrier()

    def _pref(b, ci):
        pltpu.async_copy(edges_hbm.at[wid, ci], eds[b], sps[b])
        pltpu.async_copy(attr_hbm.at[wid, ci], ats[b], sps[b])

    def _pref_wait(b):
        pltpu.make_async_copy(edges_hbm.at[wid, 0], eds[b], sps[b]).wait()
        pltpu.make_async_copy(attr_hbm.at[wid, 0], ats[b], sps[b]).wait()

    def _gather(b):
        pltpu.async_copy(h_hbm.at[eds[b].at[0]], msgs[b], sgs[b])

    def _gather_wait(b):
        pltpu.make_async_copy(h_hbm.at[eds[b].at[0]], msgs[b], sgs[b]).wait()

    def _scat(b):
        pltpu.async_copy(msgs[b], acc_sh.at[dvs[b]], sss[b], add=True)

    def _scat_wait(b):
        pltpu.make_async_copy(msgs[b], acc_sh.at[dvs[b]], sss[b]).wait()

    def _stage(b, ci, first, last):
        nb = 1 - b
        _gather_wait(b)
        for g in range(_CH // 16):
            sl = pl.ds(g * 16, 16)
            dvs[b][sl] = eds[b][1, sl]
        def _grp(gi, cc):
            av = ats[b][0, pl.ds(gi * 16, 16)]
            for j in range(16):
                a = av[j]
                e = gi * 16 + j
                for k in range(_D // 16):
                    sl = pl.ds(k * 16, 16)
                    msgs[b][e, sl] = msgs[b][e, sl] * a
            return cc
        lax.fori_loop(0, _CH // 16, _grp, 0)
        if not last:
            @pl.when(ci + 2 < _NCHUNK)
            def _():
                _pref(b, ci + 2)
        _scat(b)
        if not last:
            @pl.when(ci + 1 < _NCHUNK)
            def _():
                _pref_wait(nb)
                if not first:
                    _scat_wait(nb)
                _gather(nb)

    _pref(0, 0)
    _pref(1, 1)
    _pref_wait(0)
    _gather(0)

    def _super(i, c):
        _stage(0, 2 * i, False, False)
        _stage(1, 2 * i + 1, False, False)
        return c

    _stage(0, 0, True, False)
    _stage(1, 1, False, False)
    lax.fori_loop(1, _NCHUNK // 2, _super, 0)
    _stage(0, _NCHUNK - 1, False, True)
    _scat_wait(1)
    _scat_wait(0)
    plsc.subcore_barrier()

    for t in range(-(-_NRCH // _NS)):
        rc = sid + t * _NS
        @pl.when(rc < _NRCH)
        def _():
            pltpu.sync_copy(acc_sh.at[pl.ds(rc * _RCH, _RCH)],
                            out_hbm.at[cid, pl.ds(rc * _RCH, _RCH)])


_sc_spmm = pl.kernel(
    _sc_spmm_body,
    out_type=jax.ShapeDtypeStruct((_NC, _N, _D), jnp.float32),
    mesh=plsc.VectorSubcoreMesh(core_axis_name="c", subcore_axis_name="s"),
    scratch_types=[
        pltpu.VMEM((2, _CH), jnp.int32),
        pltpu.VMEM((2, _CH), jnp.int32),
        pltpu.VMEM((1, _CH), jnp.float32),
        pltpu.VMEM((1, _CH), jnp.float32),
        pltpu.VMEM((_CH,), jnp.int32),
        pltpu.VMEM((_CH,), jnp.int32),
        pltpu.VMEM((_CH, _D), jnp.float32),
        pltpu.VMEM((_CH, _D), jnp.float32),
        pltpu.VMEM_SHARED((_N, _D), jnp.float32),
        pltpu.SemaphoreType.DMA,
        pltpu.SemaphoreType.DMA,
        pltpu.SemaphoreType.DMA,
        pltpu.SemaphoreType.DMA,
        pltpu.SemaphoreType.DMA,
        pltpu.SemaphoreType.DMA,
    ],
)


_RB = 2000
_GRID = _N // _RB


def _linear_body(x_ref, w_ref, b_ref, o_ref):
    o_ref[...] = (jnp.dot(x_ref[...], w_ref[...],
                          preferred_element_type=jnp.float32) + b_ref[...])


def _tc_linear(x, w, b):
    return pl.pallas_call(
        _linear_body,
        grid=(_GRID,),
        in_specs=[pl.BlockSpec((_RB, _D), lambda i: (i, 0)),
                  pl.BlockSpec((_D, _D), lambda i: (0, 0)),
                  pl.BlockSpec((1, _D), lambda i: (0, 0))],
        out_specs=pl.BlockSpec((_RB, _D), lambda i: (i, 0)),
        out_shape=jax.ShapeDtypeStruct((_N, _D), jnp.float32),
    )(x, w, b.reshape(1, _D))


def _mlp(g, w1, b1, w2, b2):
    t = jnp.maximum(jnp.dot(g, w1, preferred_element_type=jnp.float32) + b1,
                    0.0)
    return jnp.dot(t, w2, preferred_element_type=jnp.float32) + b2


def _res_body(g_ref, h_ref, w1_ref, b1_ref, w2_ref, b2_ref, o_ref):
    g = g_ref[0] + g_ref[1]
    o_ref[...] = h_ref[...] + _mlp(g, w1_ref[...], b1_ref[...],
                                   w2_ref[...], b2_ref[...])


def _plain_body(g_ref, w1_ref, b1_ref, w2_ref, b2_ref, o_ref):
    g = g_ref[0] + g_ref[1]
    o_ref[...] = _mlp(g, w1_ref[...], b1_ref[...], w2_ref[...], b2_ref[...])


def _dual_body(g_ref, h_ref, w1a_ref, b1a_ref, w2a_ref, b2a_ref,
               w1b_ref, b1b_ref, w2b_ref, b2b_ref, s_ref, hn_ref):
    g = g_ref[0] + g_ref[1]
    s_ref[...] = _mlp(g, w1a_ref[...], b1a_ref[...], w2a_ref[...], b2a_ref[...])
    hn_ref[...] = h_ref[...] + _mlp(g, w1b_ref[...], b1b_ref[...],
                                    w2b_ref[...], b2b_ref[...])


_g_spec = pl.BlockSpec((_NC, _RB, _D), lambda i: (0, i, 0))
_row_spec = pl.BlockSpec((_RB, _D), lambda i: (i, 0))
_w_spec = pl.BlockSpec((_D, _D), lambda i: (0, 0))
_b_spec = pl.BlockSpec((1, _D), lambda i: (0, 0))
_row_shape = jax.ShapeDtypeStruct((_N, _D), jnp.float32)


def _tc_res(g, h, w1, b1, w2, b2):
    return pl.pallas_call(
        _res_body,
        grid=(_GRID,),
        in_specs=[_g_spec, _row_spec, _w_spec, _b_spec, _w_spec, _b_spec],
        out_specs=_row_spec,
        out_shape=_row_shape,
    )(g, h, w1, b1.reshape(1, _D), w2, b2.reshape(1, _D))


def _tc_plain(g, w1, b1, w2, b2):
    return pl.pallas_call(
        _plain_body,
        grid=(_GRID,),
        in_specs=[_g_spec, _w_spec, _b_spec, _w_spec, _b_spec],
        out_specs=_row_spec,
        out_shape=_row_shape,
    )(g, w1, b1.reshape(1, _D), w2, b2.reshape(1, _D))


def _tc_dual(g, h, w1a, b1a, w2a, b2a, w1b, b1b, w2b, b2b):
    return pl.pallas_call(
        _dual_body,
        grid=(_GRID,),
        in_specs=[_g_spec, _row_spec,
                  _w_spec, _b_spec, _w_spec, _b_spec,
                  _w_spec, _b_spec, _w_spec, _b_spec],
        out_specs=(_row_spec, _row_spec),
        out_shape=(_row_shape, _row_shape),
    )(g, h, w1a, b1a.reshape(1, _D), w2a, b2a.reshape(1, _D),
      w1b, b1b.reshape(1, _D), w2b, b2b.reshape(1, _D))


def kernel(x, edge_index, batch, scatter_edge_index, scatter_edge_attr,
           W_in, b_in, W1, b1, W2, b2):
    edges = jnp.stack(
        [scatter_edge_index[0].reshape(_NW, _NCHUNK, _CH),
         scatter_edge_index[1].reshape(_NW, _NCHUNK, _CH)], axis=2)
    attr = scatter_edge_attr.reshape(_NW, _NCHUNK, 1, _CH)

    h1 = _tc_linear(x, W_in, b_in)
    g1 = _sc_spmm(h1, edges, attr)
    h2 = _tc_res(g1, h1, W1[0], b1[0], W2[0], b2[0])
    g2 = _sc_spmm(h2, edges, attr)
    s0, h3 = _tc_dual(g2, h2, W1[0], b1[0], W2[0], b2[0],
                      W1[1], b1[1], W2[1], b2[1])
    g3 = _sc_spmm(h3, edges, attr)
    s1, h4 = _tc_dual(g3, h3, W1[1], b1[1], W2[1], b2[1],
                      W1[2], b1[2], W2[2], b2[2])
    g4 = _sc_spmm(h4, edges, attr)
    s2 = _tc_plain(g4, W1[2], b1[2], W2[2], b2[2])
    return ((s0, s1, s2), (h2, h3, h4))


# speedup vs baseline: 8.9107x; 1.2625x over previous
import functools

import jax
import jax.numpy as jnp
from jax import lax
from jax.experimental import pallas as pl
from jax.experimental.pallas import tpu as pltpu
from jax.experimental.pallas import tpu_sc as plsc

_N = 10000
_E = 320000
_D = 128

_NC = 2
_NS = 16
_NW = _NC * _NS
_CH = 80
_EPT = _E // _NW
_NCHUNK = _EPT // _CH
_RCH = 80
_NRCH = _N // _RCH


def _sc_spmm_body(h_hbm, edges_hbm, attr_hbm, out_hbm,
                  ed0, ed1, at0, at1, dv0, dv1, msg0, msg1, acc_sh,
                  sg0, sg1, ss0, ss1, sp0, sp1):
    cid = lax.axis_index("c")
    sid = lax.axis_index("s")
    wid = cid * _NS + sid

    eds = (ed0, ed1)
    ats = (at0, at1)
    dvs = (dv0, dv1)
    msgs = (msg0, msg1)
    sgs = (sg0, sg1)
    sss = (ss0, ss1)
    sps = (sp0, sp1)

    def _zrow(i, c):
        for k in range(_D // 16):
            msg0[i, pl.ds(k * 16, 16)] = jnp.zeros((16,), jnp.float32)
        return c
    lax.fori_loop(0, _RCH, _zrow, 0)

    for t in range(-(-_NRCH // _NS)):
        rc = sid + t * _NS
        @pl.when(rc < _NRCH)
        def _():
            pltpu.sync_copy(msg0, acc_sh.at[pl.ds(rc * _RCH, _RCH)])
    plsc.subcore_barrier()

    def _pref(b, ci):
        pltpu.async_copy(edges_hbm.at[wid, ci], eds[b], sps[b])
        pltpu.async_copy(attr_hbm.at[wid, ci], ats[b], sps[b])

    def _pref_wait(b):
        pltpu.make_async_copy(edges_hbm.at[wid, 0], eds[b], sps[b]).wait()
        pltpu.make_async_copy(attr_hbm.at[wid, 0], ats[b], sps[b]).wait()

    def _gather(b):
        pltpu.async_copy(h_hbm.at[eds[b].at[0]], msgs[b], sgs[b])

    def _gather_wait(b):
        pltpu.make_async_copy(h_hbm.at[eds[b].at[0]], msgs[b], sgs[b]).wait()

    def _scat(b):
        pltpu.async_copy(msgs[b], acc_sh.at[dvs[b]], sss[b], add=True)

    def _scat_wait(b):
        pltpu.make_async_copy(msgs[b], acc_sh.at[dvs[b]], sss[b]).wait()

    def _stage(b, ci, first, last):
        nb = 1 - b
        _gather_wait(b)
        if not last:
            @pl.when(ci + 1 < _NCHUNK)
            def _():
                _pref_wait(nb)
                if not first:
                    _scat_wait(nb)
                _gather(nb)
        for g in range(_CH // 16):
            sl = pl.ds(g * 16, 16)
            dvs[b][sl] = eds[b][1, sl]
        def _grp(gi, cc):
            av = ats[b][0, pl.ds(gi * 16, 16)]
            for j in range(16):
                a = av[j]
                e = gi * 16 + j
                for k in range(_D // 16):
                    sl = pl.ds(k * 16, 16)
                    msgs[b][e, sl] = msgs[b][e, sl] * a
            return cc
        lax.fori_loop(0, _CH // 16, _grp, 0)
        if not last:
            @pl.when(ci + 2 < _NCHUNK)
            def _():
                _pref(b, ci + 2)
        _scat(b)

    _pref(0, 0)
    _pref(1, 1)
    _pref_wait(0)
    _gather(0)

    def _super(i, c):
        _stage(0, 2 * i, False, False)
        _stage(1, 2 * i + 1, False, False)
        return c

    _stage(0, 0, True, False)
    _stage(1, 1, False, False)
    lax.fori_loop(1, _NCHUNK // 2, _super, 0)
    _stage(0, _NCHUNK - 1, False, True)
    _scat_wait(1)
    _scat_wait(0)
    plsc.subcore_barrier()

    for t in range(-(-_NRCH // _NS)):
        rc = sid + t * _NS
        @pl.when(rc < _NRCH)
        def _():
            pltpu.sync_copy(acc_sh.at[pl.ds(rc * _RCH, _RCH)],
                            out_hbm.at[cid, pl.ds(rc * _RCH, _RCH)])


_sc_spmm = pl.kernel(
    _sc_spmm_body,
    out_type=jax.ShapeDtypeStruct((_NC, _N, _D), jnp.float32),
    mesh=plsc.VectorSubcoreMesh(core_axis_name="c", subcore_axis_name="s"),
    scratch_types=[
        pltpu.VMEM((2, _CH), jnp.int32),
        pltpu.VMEM((2, _CH), jnp.int32),
        pltpu.VMEM((1, _CH), jnp.float32),
        pltpu.VMEM((1, _CH), jnp.float32),
        pltpu.VMEM((_CH,), jnp.int32),
        pltpu.VMEM((_CH,), jnp.int32),
        pltpu.VMEM((_CH, _D), jnp.float32),
        pltpu.VMEM((_CH, _D), jnp.float32),
        pltpu.VMEM_SHARED((_N, _D), jnp.float32),
        pltpu.SemaphoreType.DMA,
        pltpu.SemaphoreType.DMA,
        pltpu.SemaphoreType.DMA,
        pltpu.SemaphoreType.DMA,
        pltpu.SemaphoreType.DMA,
        pltpu.SemaphoreType.DMA,
    ],
)


_RB = 2000
_GRID = _N // _RB


def _linear_body(x_ref, w_ref, b_ref, o_ref):
    o_ref[...] = (jnp.dot(x_ref[...], w_ref[...],
                          preferred_element_type=jnp.float32) + b_ref[...])


def _tc_linear(x, w, b):
    return pl.pallas_call(
        _linear_body,
        grid=(_GRID,),
        in_specs=[pl.BlockSpec((_RB, _D), lambda i: (i, 0)),
                  pl.BlockSpec((_D, _D), lambda i: (0, 0)),
                  pl.BlockSpec((1, _D), lambda i: (0, 0))],
        out_specs=pl.BlockSpec((_RB, _D), lambda i: (i, 0)),
        out_shape=jax.ShapeDtypeStruct((_N, _D), jnp.float32),
    )(x, w, b.reshape(1, _D))


def _mlp(g, w1, b1, w2, b2):
    t = jnp.maximum(jnp.dot(g, w1, preferred_element_type=jnp.float32) + b1,
                    0.0)
    return jnp.dot(t, w2, preferred_element_type=jnp.float32) + b2


def _res_body(g_ref, h_ref, w1_ref, b1_ref, w2_ref, b2_ref, o_ref):
    g = g_ref[0] + g_ref[1]
    o_ref[...] = h_ref[...] + _mlp(g, w1_ref[...], b1_ref[...],
                                   w2_ref[...], b2_ref[...])


def _plain_body(g_ref, w1_ref, b1_ref, w2_ref, b2_ref, o_ref):
    g = g_ref[0] + g_ref[1]
    o_ref[...] = _mlp(g, w1_ref[...], b1_ref[...], w2_ref[...], b2_ref[...])


def _dual_body(g_ref, h_ref, w1a_ref, b1a_ref, w2a_ref, b2a_ref,
               w1b_ref, b1b_ref, w2b_ref, b2b_ref, s_ref, hn_ref):
    g = g_ref[0] + g_ref[1]
    s_ref[...] = _mlp(g, w1a_ref[...], b1a_ref[...], w2a_ref[...], b2a_ref[...])
    hn_ref[...] = h_ref[...] + _mlp(g, w1b_ref[...], b1b_ref[...],
                                    w2b_ref[...], b2b_ref[...])


_g_spec = pl.BlockSpec((_NC, _RB, _D), lambda i: (0, i, 0))
_row_spec = pl.BlockSpec((_RB, _D), lambda i: (i, 0))
_w_spec = pl.BlockSpec((_D, _D), lambda i: (0, 0))
_b_spec = pl.BlockSpec((1, _D), lambda i: (0, 0))
_row_shape = jax.ShapeDtypeStruct((_N, _D), jnp.float32)


def _tc_res(g, h, w1, b1, w2, b2):
    return pl.pallas_call(
        _res_body,
        grid=(_GRID,),
        in_specs=[_g_spec, _row_spec, _w_spec, _b_spec, _w_spec, _b_spec],
        out_specs=_row_spec,
        out_shape=_row_shape,
    )(g, h, w1, b1.reshape(1, _D), w2, b2.reshape(1, _D))


def _tc_plain(g, w1, b1, w2, b2):
    return pl.pallas_call(
        _plain_body,
        grid=(_GRID,),
        in_specs=[_g_spec, _w_spec, _b_spec, _w_spec, _b_spec],
        out_specs=_row_spec,
        out_shape=_row_shape,
    )(g, w1, b1.reshape(1, _D), w2, b2.reshape(1, _D))


def _tc_dual(g, h, w1a, b1a, w2a, b2a, w1b, b1b, w2b, b2b):
    return pl.pallas_call(
        _dual_body,
        grid=(_GRID,),
        in_specs=[_g_spec, _row_spec,
                  _w_spec, _b_spec, _w_spec, _b_spec,
                  _w_spec, _b_spec, _w_spec, _b_spec],
        out_specs=(_row_spec, _row_spec),
        out_shape=(_row_shape, _row_shape),
    )(g, h, w1a, b1a.reshape(1, _D), w2a, b2a.reshape(1, _D),
      w1b, b1b.reshape(1, _D), w2b, b2b.reshape(1, _D))


def kernel(x, edge_index, batch, scatter_edge_index, scatter_edge_attr,
           W_in, b_in, W1, b1, W2, b2):
    edges = jnp.stack(
        [scatter_edge_index[0].reshape(_NW, _NCHUNK, _CH),
         scatter_edge_index[1].reshape(_NW, _NCHUNK, _CH)], axis=2)
    attr = scatter_edge_attr.reshape(_NW, _NCHUNK, 1, _CH)

    h1 = _tc_linear(x, W_in, b_in)
    g1 = _sc_spmm(h1, edges, attr)
    h2 = _tc_res(g1, h1, W1[0], b1[0], W2[0], b2[0])
    g2 = _sc_spmm(h2, edges, attr)
    s0, h3 = _tc_dual(g2, h2, W1[0], b1[0], W2[0], b2[0],
                      W1[1], b1[1], W2[1], b2[1])
    g3 = _sc_spmm(h3, edges, attr)
    s1, h4 = _tc_dual(g3, h3, W1[1], b1[1], W2[1], b2[1],
                      W1[2], b1[2], W2[2], b2[2])
    g4 = _sc_spmm(h4, edges, attr)
    s2 = _tc_plain(g4, W1[2], b1[2], W2[2], b2[2])
    return ((s0, s1, s2), (h2, h3, h4))


# speedup vs baseline: 8.9619x; 1.0057x over previous
import functools

import jax
import jax.numpy as jnp
from jax import lax
from jax.experimental import pallas as pl
from jax.experimental.pallas import tpu as pltpu
from jax.experimental.pallas import tpu_sc as plsc

_N = 10000
_E = 320000
_D = 128

_NC = 2
_NS = 16
_NW = _NC * _NS
_CH = 80
_EPT = _E // _NW
_NCHUNK = _EPT // _CH
_RCH = 80
_NRCH = _N // _RCH


def _sc_spmm_body(h_hbm, edges_hbm, attr_hbm, out_hbm,
                  ed0, ed1, ed2, at0, at1, at2, dv0, dv1, dv2,
                  msg0, msg1, msg2, acc_sh,
                  sg0, sg1, sg2, ss0, ss1, ss2, sp0, sp1, sp2):
    cid = lax.axis_index("c")
    sid = lax.axis_index("s")
    wid = cid * _NS + sid

    eds = (ed0, ed1, ed2)
    ats = (at0, at1, at2)
    dvs = (dv0, dv1, dv2)
    msgs = (msg0, msg1, msg2)
    sgs = (sg0, sg1, sg2)
    sss = (ss0, ss1, ss2)
    sps = (sp0, sp1, sp2)

    def _zrow(i, c):
        for k in range(_D // 16):
            msg0[i, pl.ds(k * 16, 16)] = jnp.zeros((16,), jnp.float32)
        return c
    lax.fori_loop(0, _RCH, _zrow, 0)

    for t in range(-(-_NRCH // _NS)):
        rc = sid + t * _NS
        @pl.when(rc < _NRCH)
        def _():
            pltpu.sync_copy(msg0, acc_sh.at[pl.ds(rc * _RCH, _RCH)])
    plsc.subcore_barrier()

    def _pref(b, ci):
        pltpu.async_copy(edges_hbm.at[wid, ci], eds[b], sps[b])
        pltpu.async_copy(attr_hbm.at[wid, ci], ats[b], sps[b])

    def _pref_wait(b):
        pltpu.make_async_copy(edges_hbm.at[wid, 0], eds[b], sps[b]).wait()
        pltpu.make_async_copy(attr_hbm.at[wid, 0], ats[b], sps[b]).wait()

    def _gather(b):
        pltpu.async_copy(h_hbm.at[eds[b].at[0]], msgs[b], sgs[b])

    def _gather_wait(b):
        pltpu.make_async_copy(h_hbm.at[eds[b].at[0]], msgs[b], sgs[b]).wait()

    def _scat(b):
        pltpu.async_copy(msgs[b], acc_sh.at[dvs[b]], sss[b], add=True)

    def _scat_wait(b):
        pltpu.make_async_copy(msgs[b], acc_sh.at[dvs[b]], sss[b]).wait()

    def _stage(b, ci, wait_scat):
        nb = (b + 1) % 3
        _gather_wait(b)
        @pl.when(ci + 1 < _NCHUNK)
        def _():
            _pref_wait(nb)
            if wait_scat:
                _scat_wait(nb)
            _gather(nb)
        for g in range(_CH // 16):
            sl = pl.ds(g * 16, 16)
            dvs[b][sl] = eds[b][1, sl]
        def _grp(gi, cc):
            av = ats[b][0, pl.ds(gi * 16, 16)]
            for j in range(16):
                a = av[j]
                e = gi * 16 + j
                for k in range(_D // 16):
                    sl = pl.ds(k * 16, 16)
                    msgs[b][e, sl] = msgs[b][e, sl] * a
            return cc
        lax.fori_loop(0, _CH // 16, _grp, 0)
        @pl.when(ci + 3 < _NCHUNK)
        def _():
            _pref(b, ci + 3)
        _scat(b)

    _pref(0, 0)
    _pref(1, 1)
    _pref(2, 2)
    _pref_wait(0)
    _gather(0)

    _stage(0, 0, False)
    _stage(1, 1, False)

    def _super(i, c):
        ci = 3 * i + 2
        _stage(2, ci, True)
        _stage(0, ci + 1, True)
        _stage(1, ci + 2, True)
        return c
    lax.fori_loop(0, (_NCHUNK - 2) // 3, _super, 0)
    _scat_wait(2)
    _scat_wait(0)
    _scat_wait(1)
    plsc.subcore_barrier()

    for t in range(-(-_NRCH // _NS)):
        rc = sid + t * _NS
        @pl.when(rc < _NRCH)
        def _():
            pltpu.sync_copy(acc_sh.at[pl.ds(rc * _RCH, _RCH)],
                            out_hbm.at[cid, pl.ds(rc * _RCH, _RCH)])


_sc_spmm = pl.kernel(
    _sc_spmm_body,
    out_type=jax.ShapeDtypeStruct((_NC, _N, _D), jnp.float32),
    mesh=plsc.VectorSubcoreMesh(core_axis_name="c", subcore_axis_name="s"),
    scratch_types=(
        [pltpu.VMEM((2, _CH), jnp.int32)] * 3
        + [pltpu.VMEM((1, _CH), jnp.float32)] * 3
        + [pltpu.VMEM((_CH,), jnp.int32)] * 3
        + [pltpu.VMEM((_CH, _D), jnp.float32)] * 3
        + [pltpu.VMEM_SHARED((_N, _D), jnp.float32)]
        + [pltpu.SemaphoreType.DMA] * 9
    ),
)


_RB = 2000
_GRID = _N // _RB


def _linear_body(x_ref, w_ref, b_ref, o_ref):
    o_ref[...] = (jnp.dot(x_ref[...], w_ref[...],
                          preferred_element_type=jnp.float32) + b_ref[...])


def _tc_linear(x, w, b):
    return pl.pallas_call(
        _linear_body,
        grid=(_GRID,),
        in_specs=[pl.BlockSpec((_RB, _D), lambda i: (i, 0)),
                  pl.BlockSpec((_D, _D), lambda i: (0, 0)),
                  pl.BlockSpec((1, _D), lambda i: (0, 0))],
        out_specs=pl.BlockSpec((_RB, _D), lambda i: (i, 0)),
        out_shape=jax.ShapeDtypeStruct((_N, _D), jnp.float32),
    )(x, w, b.reshape(1, _D))


def _mlp(g, w1, b1, w2, b2):
    t = jnp.maximum(jnp.dot(g, w1, preferred_element_type=jnp.float32) + b1,
                    0.0)
    return jnp.dot(t, w2, preferred_element_type=jnp.float32) + b2


def _res_body(g_ref, h_ref, w1_ref, b1_ref, w2_ref, b2_ref, o_ref):
    g = g_ref[0] + g_ref[1]
    o_ref[...] = h_ref[...] + _mlp(g, w1_ref[...], b1_ref[...],
                                   w2_ref[...], b2_ref[...])


def _plain_body(g_ref, w1_ref, b1_ref, w2_ref, b2_ref, o_ref):
    g = g_ref[0] + g_ref[1]
    o_ref[...] = _mlp(g, w1_ref[...], b1_ref[...], w2_ref[...], b2_ref[...])


def _dual_body(g_ref, h_ref, w1a_ref, b1a_ref, w2a_ref, b2a_ref,
               w1b_ref, b1b_ref, w2b_ref, b2b_ref, s_ref, hn_ref):
    g = g_ref[0] + g_ref[1]
    s_ref[...] = _mlp(g, w1a_ref[...], b1a_ref[...], w2a_ref[...], b2a_ref[...])
    hn_ref[...] = h_ref[...] + _mlp(g, w1b_ref[...], b1b_ref[...],
                                    w2b_ref[...], b2b_ref[...])


_g_spec = pl.BlockSpec((_NC, _RB, _D), lambda i: (0, i, 0))
_row_spec = pl.BlockSpec((_RB, _D), lambda i: (i, 0))
_w_spec = pl.BlockSpec((_D, _D), lambda i: (0, 0))
_b_spec = pl.BlockSpec((1, _D), lambda i: (0, 0))
_row_shape = jax.ShapeDtypeStruct((_N, _D), jnp.float32)


def _tc_res(g, h, w1, b1, w2, b2):
    return pl.pallas_call(
        _res_body,
        grid=(_GRID,),
        in_specs=[_g_spec, _row_spec, _w_spec, _b_spec, _w_spec, _b_spec],
        out_specs=_row_spec,
        out_shape=_row_shape,
    )(g, h, w1, b1.reshape(1, _D), w2, b2.reshape(1, _D))


def _tc_plain(g, w1, b1, w2, b2):
    return pl.pallas_call(
        _plain_body,
        grid=(_GRID,),
        in_specs=[_g_spec, _w_spec, _b_spec, _w_spec, _b_spec],
        out_specs=_row_spec,
        out_shape=_row_shape,
    )(g, w1, b1.reshape(1, _D), w2, b2.reshape(1, _D))


def _tc_dual(g, h, w1a, b1a, w2a, b2a, w1b, b1b, w2b, b2b):
    return pl.pallas_call(
        _dual_body,
        grid=(_GRID,),
        in_specs=[_g_spec, _row_spec,
                  _w_spec, _b_spec, _w_spec, _b_spec,
                  _w_spec, _b_spec, _w_spec, _b_spec],
        out_specs=(_row_spec, _row_spec),
        out_shape=(_row_shape, _row_shape),
    )(g, h, w1a, b1a.reshape(1, _D), w2a, b2a.reshape(1, _D),
      w1b, b1b.reshape(1, _D), w2b, b2b.reshape(1, _D))


def kernel(x, edge_index, batch, scatter_edge_index, scatter_edge_attr,
           W_in, b_in, W1, b1, W2, b2):
    edges = jnp.stack(
        [scatter_edge_index[0].reshape(_NW, _NCHUNK, _CH),
         scatter_edge_index[1].reshape(_NW, _NCHUNK, _CH)], axis=2)
    attr = scatter_edge_attr.reshape(_NW, _NCHUNK, 1, _CH)

    h1 = _tc_linear(x, W_in, b_in)
    g1 = _sc_spmm(h1, edges, attr)
    h2 = _tc_res(g1, h1, W1[0], b1[0], W2[0], b2[0])
    g2 = _sc_spmm(h2, edges, attr)
    s0, h3 = _tc_dual(g2, h2, W1[0], b1[0], W2[0], b2[0],
                      W1[1], b1[1], W2[1], b2[1])
    g3 = _sc_spmm(h3, edges, attr)
    s1, h4 = _tc_dual(g3, h3, W1[1], b1[1], W2[1], b2[1],
                      W1[2], b1[2], W2[2], b2[2])
    g4 = _sc_spmm(h4, edges, attr)
    s2 = _tc_plain(g4, W1[2], b1[2], W2[2], b2[2])
    return ((s0, s1, s2), (h2, h3, h4))


# speedup vs baseline: 8.9687x; 1.0008x over previous
import functools

import jax
import jax.numpy as jnp
from jax import lax
from jax.experimental import pallas as pl
from jax.experimental.pallas import tpu as pltpu
from jax.experimental.pallas import tpu_sc as plsc

_N = 10000
_E = 320000
_D = 128

_NC = 2
_NS = 16
_NW = _NC * _NS
_CH = 80
_EPT = _E // _NW
_NCHUNK = _EPT // _CH
_RCH = 80
_NRCH = _N // _RCH


def _sc_spmm_body(h_hbm, edges_hbm, attr_hbm, out_hbm,
                  ed0, ed1, ed2, at0, at1, at2, dv0, dv1, dv2,
                  msg0, msg1, msg2, acc_sh,
                  sg0, sg1, sg2, ss0, ss1, ss2, sp0, sp1, sp2):
    cid = lax.axis_index("c")
    sid = lax.axis_index("s")
    wid = cid * _NS + sid

    eds = (ed0, ed1, ed2)
    ats = (at0, at1, at2)
    dvs = (dv0, dv1, dv2)
    msgs = (msg0, msg1, msg2)
    sgs = (sg0, sg1, sg2)
    sss = (ss0, ss1, ss2)
    sps = (sp0, sp1, sp2)

    def _zrow(i, c):
        for k in range(_D // 16):
            msg0[i, pl.ds(k * 16, 16)] = jnp.zeros((16,), jnp.float32)
        return c
    lax.fori_loop(0, _RCH, _zrow, 0)

    for t in range(-(-_NRCH // _NS)):
        rc = sid + t * _NS
        @pl.when(rc < _NRCH)
        def _():
            pltpu.sync_copy(msg0, acc_sh.at[pl.ds(rc * _RCH, _RCH)])
    plsc.subcore_barrier()

    def _pref(b, ci):
        pltpu.async_copy(edges_hbm.at[wid, ci], eds[b], sps[b])
        pltpu.async_copy(attr_hbm.at[wid, ci], ats[b], sps[b])

    def _pref_wait(b):
        pltpu.make_async_copy(edges_hbm.at[wid, 0], eds[b], sps[b]).wait()
        pltpu.make_async_copy(attr_hbm.at[wid, 0], ats[b], sps[b]).wait()

    def _gather(b):
        pltpu.async_copy(h_hbm.at[eds[b].at[0]], msgs[b], sgs[b])

    def _gather_wait(b):
        pltpu.make_async_copy(h_hbm.at[eds[b].at[0]], msgs[b], sgs[b]).wait()

    def _scat(b):
        pltpu.async_copy(msgs[b], acc_sh.at[dvs[b]], sss[b], add=True)

    def _scat_wait(b):
        pltpu.make_async_copy(msgs[b], acc_sh.at[dvs[b]], sss[b]).wait()

    def _stage(b, ci, wait_scat):
        nb = (b + 1) % 3
        _gather_wait(b)
        @pl.when(ci + 1 < _NCHUNK)
        def _():
            _pref_wait(nb)
            if wait_scat:
                _scat_wait(nb)
            _gather(nb)
        for g in range(_CH // 16):
            sl = pl.ds(g * 16, 16)
            dvs[b][sl] = eds[b][1, sl]
        def _grp(gi, cc):
            av = ats[b][0, pl.ds(gi * 16, 16)]
            for j in range(16):
                a = av[j]
                e = gi * 16 + j
                for k in range(_D // 16):
                    sl = pl.ds(k * 16, 16)
                    msgs[b][e, sl] = msgs[b][e, sl] * a
            return cc
        lax.fori_loop(0, _CH // 16, _grp, 0)
        @pl.when(ci + 3 < _NCHUNK)
        def _():
            _pref(b, ci + 3)
        _scat(b)

    _pref(0, 0)
    _pref(1, 1)
    _pref(2, 2)
    _pref_wait(0)
    _gather(0)

    _stage(0, 0, False)
    _stage(1, 1, False)

    def _super(i, c):
        ci = 3 * i + 2
        _stage(2, ci, True)
        _stage(0, ci + 1, True)
        _stage(1, ci + 2, True)
        return c
    lax.fori_loop(0, (_NCHUNK - 2) // 3, _super, 0)
    _scat_wait(2)
    _scat_wait(0)
    _scat_wait(1)
    plsc.subcore_barrier()

    for t in range(-(-_NRCH // _NS)):
        rc = sid + t * _NS
        @pl.when(rc < _NRCH)
        def _():
            pltpu.sync_copy(acc_sh.at[pl.ds(rc * _RCH, _RCH)],
                            out_hbm.at[cid, pl.ds(rc * _RCH, _RCH)])


_sc_spmm = pl.kernel(
    _sc_spmm_body,
    out_type=jax.ShapeDtypeStruct((_NC, _N, _D), jnp.float32),
    mesh=plsc.VectorSubcoreMesh(core_axis_name="c", subcore_axis_name="s"),
    scratch_types=(
        [pltpu.VMEM((2, _CH), jnp.int32)] * 3
        + [pltpu.VMEM((1, _CH), jnp.float32)] * 3
        + [pltpu.VMEM((_CH,), jnp.int32)] * 3
        + [pltpu.VMEM((_CH, _D), jnp.float32)] * 3
        + [pltpu.VMEM_SHARED((_N, _D), jnp.float32)]
        + [pltpu.SemaphoreType.DMA] * 9
    ),
)


_RB = 2000
_GRID = _N // _RB


def _linear_body(x_ref, w_ref, b_ref, o_ref):
    o_ref[...] = (jnp.dot(x_ref[...], w_ref[...],
                          preferred_element_type=jnp.float32) + b_ref[...])


def _tc_linear(x, w, b):
    return pl.pallas_call(
        _linear_body,
        grid=(_GRID,),
        in_specs=[pl.BlockSpec((_RB, _D), lambda i: (i, 0)),
                  pl.BlockSpec((_D, _D), lambda i: (0, 0)),
                  pl.BlockSpec((1, _D), lambda i: (0, 0))],
        out_specs=pl.BlockSpec((_RB, _D), lambda i: (i, 0)),
        out_shape=jax.ShapeDtypeStruct((_N, _D), jnp.float32),
    )(x, w, b.reshape(1, _D))


def _mlp(g, w1, b1, w2, b2):
    t = jnp.maximum(jnp.dot(g, w1, preferred_element_type=jnp.float32) + b1,
                    0.0)
    return jnp.dot(t, w2, preferred_element_type=jnp.float32) + b2


def _res_body(g_ref, h_ref, w1_ref, b1_ref, w2_ref, b2_ref, o_ref):
    g = g_ref[0] + g_ref[1]
    o_ref[...] = h_ref[...] + _mlp(g, w1_ref[...], b1_ref[...],
                                   w2_ref[...], b2_ref[...])


def _plain_body(g_ref, w1_ref, b1_ref, w2_ref, b2_ref, o_ref):
    g = g_ref[0] + g_ref[1]
    o_ref[...] = _mlp(g, w1_ref[...], b1_ref[...], w2_ref[...], b2_ref[...])


def _dual_body(g_ref, h_ref, w1a_ref, b1a_ref, w2a_ref, b2a_ref,
               w1b_ref, b1b_ref, w2b_ref, b2b_ref, s_ref, hn_ref):
    g = g_ref[0] + g_ref[1]
    s_ref[...] = _mlp(g, w1a_ref[...], b1a_ref[...], w2a_ref[...], b2a_ref[...])
    hn_ref[...] = h_ref[...] + _mlp(g, w1b_ref[...], b1b_ref[...],
                                    w2b_ref[...], b2b_ref[...])


_g_spec = pl.BlockSpec((_NC, _RB, _D), lambda i: (0, i, 0))
_row_spec = pl.BlockSpec((_RB, _D), lambda i: (i, 0))
_w_spec = pl.BlockSpec((_D, _D), lambda i: (0, 0))
_b_spec = pl.BlockSpec((1, _D), lambda i: (0, 0))
_row_shape = jax.ShapeDtypeStruct((_N, _D), jnp.float32)


def _tc_res(g, h, w1, b1, w2, b2):
    return pl.pallas_call(
        _res_body,
        grid=(_GRID,),
        in_specs=[_g_spec, _row_spec, _w_spec, _b_spec, _w_spec, _b_spec],
        out_specs=_row_spec,
        out_shape=_row_shape,
    )(g, h, w1, b1.reshape(1, _D), w2, b2.reshape(1, _D))


def _tc_plain(g, w1, b1, w2, b2):
    return pl.pallas_call(
        _plain_body,
        grid=(_GRID,),
        in_specs=[_g_spec, _w_spec, _b_spec, _w_spec, _b_spec],
        out_specs=_row_spec,
        out_shape=_row_shape,
    )(g, w1, b1.reshape(1, _D), w2, b2.reshape(1, _D))


def _tc_dual(g, h, w1a, b1a, w2a, b2a, w1b, b1b, w2b, b2b):
    return pl.pallas_call(
        _dual_body,
        grid=(_GRID,),
        in_specs=[_g_spec, _row_spec,
                  _w_spec, _b_spec, _w_spec, _b_spec,
                  _w_spec, _b_spec, _w_spec, _b_spec],
        out_specs=(_row_spec, _row_spec),
        out_shape=(_row_shape, _row_shape),
    )(g, h, w1a, b1a.reshape(1, _D), w2a, b2a.reshape(1, _D),
      w1b, b1b.reshape(1, _D), w2b, b2b.reshape(1, _D))


def kernel(x, edge_index, batch, scatter_edge_index, scatter_edge_attr,
           W_in, b_in, W1, b1, W2, b2):
    edges = jnp.stack(
        [scatter_edge_index[0].reshape(_NW, _NCHUNK, _CH),
         scatter_edge_index[1].reshape(_NW, _NCHUNK, _CH)], axis=2)
    attr = scatter_edge_attr.reshape(_NW, _NCHUNK, 1, _CH)

    h1 = _tc_linear(x, W_in, b_in)
    g1 = _sc_spmm(h1, edges, attr)
    h2 = _tc_res(g1, h1, W1[0], b1[0], W2[0], b2[0])
    g2 = _sc_spmm(h2, edges, attr)
    h3 = _tc_res(g2, h2, W1[1], b1[1], W2[1], b2[1])
    g3 = _sc_spmm(h3, edges, attr)
    s0 = _tc_plain(g2, W1[0], b1[0], W2[0], b2[0])
    h4 = _tc_res(g3, h3, W1[2], b1[2], W2[2], b2[2])
    g4 = _sc_spmm(h4, edges, attr)
    s1 = _tc_plain(g3, W1[1], b1[1], W2[1], b2[1])
    s2 = _tc_plain(g4, W1[2], b1[2], W2[2], b2[2])
    return ((s0, s1, s2), (h2, h3, h4))


# speedup vs baseline: 8.9874x; 1.0021x over previous
import functools

import jax
import jax.numpy as jnp
from jax import lax
from jax.experimental import pallas as pl
from jax.experimental.pallas import tpu as pltpu
from jax.experimental.pallas import tpu_sc as plsc

_N = 10000
_E = 320000
_D = 128

_NC = 2
_NS = 16
_NW = _NC * _NS
_CH = 80
_EPT = _E // _NW
_NCHUNK = _EPT // _CH
_RCH = 80
_NRCH = _N // _RCH


def _sc_spmm_body(h_hbm, edges_hbm, attr_hbm, out_hbm,
                  ed0, ed1, ed2, at0, at1, at2, dv0, dv1, dv2,
                  msg0, msg1, msg2, acc_sh,
                  sg0, sg1, sg2, ss0, ss1, ss2, sp0, sp1, sp2):
    cid = lax.axis_index("c")
    sid = lax.axis_index("s")
    wid = cid * _NS + sid

    eds = (ed0, ed1, ed2)
    ats = (at0, at1, at2)
    dvs = (dv0, dv1, dv2)
    msgs = (msg0, msg1, msg2)
    sgs = (sg0, sg1, sg2)
    sss = (ss0, ss1, ss2)
    sps = (sp0, sp1, sp2)

    def _zrow(i, c):
        for k in range(_D // 16):
            msg0[i, pl.ds(k * 16, 16)] = jnp.zeros((16,), jnp.float32)
        return c
    lax.fori_loop(0, _RCH, _zrow, 0)

    for t in range(-(-_NRCH // _NS)):
        rc = sid + t * _NS
        @pl.when(rc < _NRCH)
        def _():
            pltpu.sync_copy(msg0, acc_sh.at[pl.ds(rc * _RCH, _RCH)])
    plsc.subcore_barrier()

    def _pref(b, ci):
        pltpu.async_copy(edges_hbm.at[wid, ci], eds[b], sps[b])
        pltpu.async_copy(attr_hbm.at[wid, ci], ats[b], sps[b])

    def _pref_wait(b):
        pltpu.make_async_copy(edges_hbm.at[wid, 0], eds[b], sps[b]).wait()
        pltpu.make_async_copy(attr_hbm.at[wid, 0], ats[b], sps[b]).wait()

    _HC = _CH // 2

    def _gather(b):
        pltpu.async_copy(h_hbm.at[eds[b].at[0, pl.ds(0, _HC)]],
                         msgs[b].at[pl.ds(0, _HC)], sgs[b])
        pltpu.async_copy(h_hbm.at[eds[b].at[0, pl.ds(_HC, _HC)]],
                         msgs[b].at[pl.ds(_HC, _HC)], sgs[b])

    def _gather_wait(b):
        pltpu.make_async_copy(h_hbm.at[eds[b].at[0, pl.ds(0, _HC)]],
                              msgs[b].at[pl.ds(0, _HC)], sgs[b]).wait()
        pltpu.make_async_copy(h_hbm.at[eds[b].at[0, pl.ds(_HC, _HC)]],
                              msgs[b].at[pl.ds(_HC, _HC)], sgs[b]).wait()

    def _scat(b):
        pltpu.async_copy(msgs[b], acc_sh.at[dvs[b]], sss[b], add=True)

    def _scat_wait(b):
        pltpu.make_async_copy(msgs[b], acc_sh.at[dvs[b]], sss[b]).wait()

    def _stage(b, ci, wait_scat):
        nb = (b + 1) % 3
        _gather_wait(b)
        @pl.when(ci + 1 < _NCHUNK)
        def _():
            _pref_wait(nb)
            if wait_scat:
                _scat_wait(nb)
            _gather(nb)
        for g in range(_CH // 16):
            sl = pl.ds(g * 16, 16)
            dvs[b][sl] = eds[b][1, sl]
        def _grp(gi, cc):
            av = ats[b][0, pl.ds(gi * 16, 16)]
            for j in range(16):
                a = av[j]
                e = gi * 16 + j
                for k in range(_D // 16):
                    sl = pl.ds(k * 16, 16)
                    msgs[b][e, sl] = msgs[b][e, sl] * a
            return cc
        lax.fori_loop(0, _CH // 16, _grp, 0)
        @pl.when(ci + 3 < _NCHUNK)
        def _():
            _pref(b, ci + 3)
        _scat(b)

    _pref(0, 0)
    _pref(1, 1)
    _pref(2, 2)
    _pref_wait(0)
    _gather(0)

    _stage(0, 0, False)
    _stage(1, 1, False)

    def _super(i, c):
        ci = 3 * i + 2
        _stage(2, ci, True)
        _stage(0, ci + 1, True)
        _stage(1, ci + 2, True)
        return c
    lax.fori_loop(0, (_NCHUNK - 2) // 3, _super, 0)
    _scat_wait(2)
    _scat_wait(0)
    _scat_wait(1)
    plsc.subcore_barrier()

    for t in range(-(-_NRCH // _NS)):
        rc = sid + t * _NS
        @pl.when(rc < _NRCH)
        def _():
            pltpu.sync_copy(acc_sh.at[pl.ds(rc * _RCH, _RCH)],
                            out_hbm.at[cid, pl.ds(rc * _RCH, _RCH)])


_sc_spmm = pl.kernel(
    _sc_spmm_body,
    out_type=jax.ShapeDtypeStruct((_NC, _N, _D), jnp.float32),
    mesh=plsc.VectorSubcoreMesh(core_axis_name="c", subcore_axis_name="s"),
    scratch_types=(
        [pltpu.VMEM((2, _CH), jnp.int32)] * 3
        + [pltpu.VMEM((1, _CH), jnp.float32)] * 3
        + [pltpu.VMEM((_CH,), jnp.int32)] * 3
        + [pltpu.VMEM((_CH, _D), jnp.float32)] * 3
        + [pltpu.VMEM_SHARED((_N, _D), jnp.float32)]
        + [pltpu.SemaphoreType.DMA] * 9
    ),
)


_RB = 2000
_GRID = _N // _RB


def _linear_body(x_ref, w_ref, b_ref, o_ref):
    o_ref[...] = (jnp.dot(x_ref[...], w_ref[...],
                          preferred_element_type=jnp.float32) + b_ref[...])


def _tc_linear(x, w, b):
    return pl.pallas_call(
        _linear_body,
        grid=(_GRID,),
        in_specs=[pl.BlockSpec((_RB, _D), lambda i: (i, 0)),
                  pl.BlockSpec((_D, _D), lambda i: (0, 0)),
                  pl.BlockSpec((1, _D), lambda i: (0, 0))],
        out_specs=pl.BlockSpec((_RB, _D), lambda i: (i, 0)),
        out_shape=jax.ShapeDtypeStruct((_N, _D), jnp.float32),
    )(x, w, b.reshape(1, _D))


def _mlp(g, w1, b1, w2, b2):
    t = jnp.maximum(jnp.dot(g, w1, preferred_element_type=jnp.float32) + b1,
                    0.0)
    return jnp.dot(t, w2, preferred_element_type=jnp.float32) + b2


def _res_body(g_ref, h_ref, w1_ref, b1_ref, w2_ref, b2_ref, o_ref):
    g = g_ref[0] + g_ref[1]
    o_ref[...] = h_ref[...] + _mlp(g, w1_ref[...], b1_ref[...],
                                   w2_ref[...], b2_ref[...])


def _plain_body(g_ref, w1_ref, b1_ref, w2_ref, b2_ref, o_ref):
    g = g_ref[0] + g_ref[1]
    o_ref[...] = _mlp(g, w1_ref[...], b1_ref[...], w2_ref[...], b2_ref[...])


def _dual_body(g_ref, h_ref, w1a_ref, b1a_ref, w2a_ref, b2a_ref,
               w1b_ref, b1b_ref, w2b_ref, b2b_ref, s_ref, hn_ref):
    g = g_ref[0] + g_ref[1]
    s_ref[...] = _mlp(g, w1a_ref[...], b1a_ref[...], w2a_ref[...], b2a_ref[...])
    hn_ref[...] = h_ref[...] + _mlp(g, w1b_ref[...], b1b_ref[...],
                                    w2b_ref[...], b2b_ref[...])


_g_spec = pl.BlockSpec((_NC, _RB, _D), lambda i: (0, i, 0))
_row_spec = pl.BlockSpec((_RB, _D), lambda i: (i, 0))
_w_spec = pl.BlockSpec((_D, _D), lambda i: (0, 0))
_b_spec = pl.BlockSpec((1, _D), lambda i: (0, 0))
_row_shape = jax.ShapeDtypeStruct((_N, _D), jnp.float32)


def _tc_res(g, h, w1, b1, w2, b2):
    return pl.pallas_call(
        _res_body,
        grid=(_GRID,),
        in_specs=[_g_spec, _row_spec, _w_spec, _b_spec, _w_spec, _b_spec],
        out_specs=_row_spec,
        out_shape=_row_shape,
    )(g, h, w1, b1.reshape(1, _D), w2, b2.reshape(1, _D))


def _tc_plain(g, w1, b1, w2, b2):
    return pl.pallas_call(
        _plain_body,
        grid=(_GRID,),
        in_specs=[_g_spec, _w_spec, _b_spec, _w_spec, _b_spec],
        out_specs=_row_spec,
        out_shape=_row_shape,
    )(g, w1, b1.reshape(1, _D), w2, b2.reshape(1, _D))


def _tc_dual(g, h, w1a, b1a, w2a, b2a, w1b, b1b, w2b, b2b):
    return pl.pallas_call(
        _dual_body,
        grid=(_GRID,),
        in_specs=[_g_spec, _row_spec,
                  _w_spec, _b_spec, _w_spec, _b_spec,
                  _w_spec, _b_spec, _w_spec, _b_spec],
        out_specs=(_row_spec, _row_spec),
        out_shape=(_row_shape, _row_shape),
    )(g, h, w1a, b1a.reshape(1, _D), w2a, b2a.reshape(1, _D),
      w1b, b1b.reshape(1, _D), w2b, b2b.reshape(1, _D))


def kernel(x, edge_index, batch, scatter_edge_index, scatter_edge_attr,
           W_in, b_in, W1, b1, W2, b2):
    edges = jnp.stack(
        [scatter_edge_index[0].reshape(_NW, _NCHUNK, _CH),
         scatter_edge_index[1].reshape(_NW, _NCHUNK, _CH)], axis=2)
    attr = scatter_edge_attr.reshape(_NW, _NCHUNK, 1, _CH)

    h1 = _tc_linear(x, W_in, b_in)
    g1 = _sc_spmm(h1, edges, attr)
    h2 = _tc_res(g1, h1, W1[0], b1[0], W2[0], b2[0])
    g2 = _sc_spmm(h2, edges, attr)
    h3 = _tc_res(g2, h2, W1[1], b1[1], W2[1], b2[1])
    g3 = _sc_spmm(h3, edges, attr)
    s0 = _tc_plain(g2, W1[0], b1[0], W2[0], b2[0])
    h4 = _tc_res(g3, h3, W1[2], b1[2], W2[2], b2[2])
    g4 = _sc_spmm(h4, edges, attr)
    s1 = _tc_plain(g3, W1[1], b1[1], W2[1], b2[1])
    s2 = _tc_plain(g4, W1[2], b1[2], W2[2], b2[2])
    return ((s0, s1, s2), (h2, h3, h4))
